# baseline (speedup 1.0000x reference)
import functools

import jax
import jax.numpy as jnp
from jax import lax
from jax.experimental import pallas as pl
from jax.experimental.pallas import tpu as pltpu
from jax.experimental.pallas import tpu_sc as plsc

_MAX_SIZE = 1000000
_STATE_DIM = 64
_BATCH = 16384

_NBLK = _MAX_SIZE // 8
_NC = 2
_NS = 16
_NW = _NC * _NS
_B_PER_W = _BATCH // _NW
_CH = 16
_NCHK = _B_PER_W // _CH


def _make_sample_kernel():
    mesh = plsc.VectorSubcoreMesh(core_axis_name="c", subcore_axis_name="s")

    @functools.partial(
        pl.kernel,
        mesh=mesh,
        compiler_params=pltpu.CompilerParams(needs_layout_passes=False),
        out_type=(
            jax.ShapeDtypeStruct((_BATCH, _STATE_DIM), jnp.float32),
            jax.ShapeDtypeStruct((_BATCH,), jnp.int32),
            jax.ShapeDtypeStruct((_BATCH,), jnp.float32),
            jax.ShapeDtypeStruct((_BATCH, _STATE_DIM), jnp.float32),
            jax.ShapeDtypeStruct((_BATCH,), jnp.float32),
        ),
        scratch_types=[
            pltpu.VMEM((_B_PER_W + 16,), jnp.int32),
            pltpu.VMEM((_CH, 8, _STATE_DIM), jnp.float32),
            pltpu.VMEM((_CH, 8, _STATE_DIM), jnp.float32),
            pltpu.VMEM((_CH, 8, 1), jnp.int32),
            pltpu.VMEM((_CH, 8, 1), jnp.float32),
            pltpu.VMEM((_CH, 8, 1), jnp.float32),
            pltpu.VMEM((_CH, _STATE_DIM), jnp.float32),
            pltpu.VMEM((_CH, _STATE_DIM), jnp.float32),
            pltpu.VMEM((_B_PER_W,), jnp.int32),
            pltpu.VMEM((_B_PER_W,), jnp.float32),
            pltpu.VMEM((_B_PER_W,), jnp.float32),
            pltpu.SemaphoreType.DMA,
            pltpu.SemaphoreType.DMA,
            pltpu.SemaphoreType.DMA,
            pltpu.SemaphoreType.DMA,
            pltpu.SemaphoreType.DMA,
        ],
    )
    def sample(s3, sn3, a3, r3, dw3, ind_hbm,
               s_out, a_out, r_out, sn_out, dw_out,
               idx_v, sbuf, snbuf, abuf, rbuf, dwbuf,
               sobuf, snobuf, aobuf, robuf, dwobuf,
               s_sem, sn_sem, a_sem, r_sem, dw_sem):
        wid = lax.axis_index("s") * _NC + lax.axis_index("c")
        base = wid * _B_PER_W
        pltpu.sync_copy(ind_hbm.at[pl.ds(base, _B_PER_W)],
                        idx_v.at[pl.ds(0, _B_PER_W)])
        zer16 = lax.iota(jnp.int32, 16) * 0
        iota16 = lax.iota(jnp.int32, 16)

        for c in range(_NCHK):
            def fire(k, _):
                i = idx_v[pl.ds(c * _CH + k, 16)][0]
                b = i // 8
                pltpu.async_copy(s3.at[b], sbuf.at[k], s_sem)
                pltpu.async_copy(sn3.at[b], snbuf.at[k], sn_sem)
                pltpu.async_copy(a3.at[b], abuf.at[k], a_sem)
                pltpu.async_copy(r3.at[b], rbuf.at[k], r_sem)
                pltpu.async_copy(dw3.at[b], dwbuf.at[k], dw_sem)
                return ()

            lax.fori_loop(0, _CH, fire, ())
            csl = pl.ds(0, _CH)
            pltpu.make_async_copy(s3.at[csl], sbuf, s_sem).wait()
            pltpu.make_async_copy(sn3.at[csl], snbuf, sn_sem).wait()
            pltpu.make_async_copy(a3.at[csl], abuf, a_sem).wait()
            pltpu.make_async_copy(r3.at[csl], rbuf, r_sem).wait()
            pltpu.make_async_copy(dw3.at[csl], dwbuf, dw_sem).wait()

            def extract_wide(k, _):
                i = idx_v[pl.ds(c * _CH + k, 16)][0]
                sub = lax.rem(i, 8)
                for j in range(_STATE_DIM // 16):
                    js = pl.ds(j * 16, 16)
                    sobuf[k, js] = sbuf[k, sub, js]
                    snobuf[k, js] = snbuf[k, sub, js]
                return ()

            lax.fori_loop(0, _CH, extract_wide, ())
            osl = pl.ds(base + c * _CH, _CH)
            pltpu.sync_copy(sobuf, s_out.at[osl])
            pltpu.sync_copy(snobuf, sn_out.at[osl])

            for g in range(_CH // 16):
                iv = idx_v[pl.ds(c * _CH + g * 16, 16)]
                kv = g * 16 + iota16
                subv = lax.rem(iv, 8)
                gsl = pl.ds(c * _CH + g * 16, 16)
                aobuf[gsl] = plsc.load_gather(abuf, [kv, subv, zer16])
                robuf[gsl] = plsc.load_gather(rbuf, [kv, subv, zer16])
                dwobuf[gsl] = plsc.load_gather(dwbuf, [kv, subv, zer16])

        nsl = pl.ds(base, _B_PER_W)
        pltpu.sync_copy(aobuf, a_out.at[nsl])
        pltpu.sync_copy(robuf, r_out.at[nsl])
        pltpu.sync_copy(dwobuf, dw_out.at[nsl])

    return sample


_sample = _make_sample_kernel()


def kernel(s, a, r, s_next, dw, ind):
    s3 = s.reshape(_NBLK, 8, _STATE_DIM)
    sn3 = s_next.reshape(_NBLK, 8, _STATE_DIM)
    a3 = a.reshape(_NBLK, 8, 1)
    r3 = r.reshape(_NBLK, 8, 1)
    dw3 = dw.reshape(_NBLK, 8, 1)
    s_b, a_b, r_b, sn_b, dw_b = _sample(s3, sn3, a3, r3, dw3, ind)
    return (s_b, a_b.reshape(_BATCH, 1), r_b.reshape(_BATCH, 1),
            sn_b, dw_b.reshape(_BATCH, 1))


# speedup vs baseline: 1.8376x; 1.8376x over previous
import functools

import jax
import jax.numpy as jnp
from jax import lax
from jax.experimental import pallas as pl
from jax.experimental.pallas import tpu as pltpu
from jax.experimental.pallas import tpu_sc as plsc

_MAX_SIZE = 1000000
_STATE_DIM = 64
_BATCH = 16384

_NC = 2
_NS = 16
_NW = _NC * _NS
_B_PER_W = _BATCH // _NW
_CHUNK = 128
_NCHUNK = _B_PER_W // _CHUNK

_mesh = plsc.VectorSubcoreMesh(core_axis_name="c", subcore_axis_name="s")


def _make_gather_kernel(ncols, dtype):
    row_shape = (_STATE_DIM,) if ncols == _STATE_DIM else ()
    out_shape = (_BATCH,) + row_shape
    buf_shape = (_B_PER_W,) + row_shape

    @functools.partial(
        pl.kernel,
        mesh=_mesh,
        compiler_params=pltpu.CompilerParams(use_tc_tiling_on_sc=False),
        out_type=(jax.ShapeDtypeStruct(out_shape, dtype),),
        scratch_types=[
            pltpu.VMEM((_NCHUNK, _CHUNK), jnp.int32),
            pltpu.VMEM(buf_shape, dtype),
            pltpu.SemaphoreType.DMA,
        ],
    )
    def gather_one(tab_hbm, ind_hbm, out, idx_v, buf, sem):
        wid = lax.axis_index("s") * _NC + lax.axis_index("c")
        base = wid * _B_PER_W
        pltpu.sync_copy(ind_hbm.at[wid], idx_v)
        copies = []
        for j in range(_NCHUNK):
            sl = pl.ds(j * _CHUNK, _CHUNK)
            copies.append(
                pltpu.async_copy(tab_hbm.at[idx_v.at[j]], buf.at[sl], sem))
        for c in copies:
            c.wait()
        pltpu.sync_copy(buf, out.at[pl.ds(base, _B_PER_W)])

    return gather_one


_gather_wide = _make_gather_kernel(_STATE_DIM, jnp.float32)
_gather_i32 = _make_gather_kernel(1, jnp.int32)
_gather_f32 = _make_gather_kernel(1, jnp.float32)


def kernel(s, a, r, s_next, dw, ind):
    ind3 = ind.reshape(_NW, _NCHUNK, _CHUNK)
    s_b, = _gather_wide(s, ind3)
    sn_b, = _gather_wide(s_next, ind3)
    a_b, = _gather_i32(a.reshape(_MAX_SIZE), ind3)
    r_b, = _gather_f32(r.reshape(_MAX_SIZE), ind3)
    dw_b, = _gather_f32(dw.reshape(_MAX_SIZE), ind3)
    return (s_b, a_b.reshape(_BATCH, 1), r_b.reshape(_BATCH, 1),
            sn_b, dw_b.reshape(_BATCH, 1))


# speedup vs baseline: 2.0471x; 1.1140x over previous
import functools

import jax
import jax.numpy as jnp
from jax import lax
from jax.experimental import pallas as pl
from jax.experimental.pallas import tpu as pltpu
from jax.experimental.pallas import tpu_sc as plsc

_MAX_SIZE = 1000000
_STATE_DIM = 64
_BATCH = 16384

_NC = 2
_NS = 16
_NW = _NC * _NS
_B_PER_W = _BATCH // _NW
_CHUNK = 128
_NCHUNK = _B_PER_W // _CHUNK


def _make_sample_kernel():
    mesh = plsc.VectorSubcoreMesh(core_axis_name="c", subcore_axis_name="s")

    @functools.partial(
        pl.kernel,
        mesh=mesh,
        compiler_params=pltpu.CompilerParams(use_tc_tiling_on_sc=False),
        out_type=(
            jax.ShapeDtypeStruct((_BATCH, _STATE_DIM), jnp.float32),
            jax.ShapeDtypeStruct((_BATCH,), jnp.int32),
            jax.ShapeDtypeStruct((_BATCH,), jnp.float32),
            jax.ShapeDtypeStruct((_BATCH, _STATE_DIM), jnp.float32),
            jax.ShapeDtypeStruct((_BATCH,), jnp.float32),
        ),
        scratch_types=[
            pltpu.VMEM((_NCHUNK, _CHUNK), jnp.int32),
            pltpu.VMEM((_B_PER_W, _STATE_DIM), jnp.float32),
            pltpu.VMEM((_B_PER_W,), jnp.int32),
            pltpu.VMEM((_B_PER_W,), jnp.float32),
            pltpu.VMEM((_B_PER_W, _STATE_DIM), jnp.float32),
            pltpu.VMEM((_B_PER_W,), jnp.float32),
            pltpu.SemaphoreType.DMA,
        ],
    )
    def sample(s_hbm, a_hbm, r_hbm, sn_hbm, dw_hbm, ind_hbm,
               s_out, a_out, r_out, sn_out, dw_out,
               idx_v, s_v, a_v, r_v, sn_v, dw_v, sem):
        wid = lax.axis_index("s") * _NC + lax.axis_index("c")
        base = wid * _B_PER_W
        pltpu.sync_copy(ind_hbm.at[wid], idx_v)
        copies = []
        for j in range(_NCHUNK):
            idx_j = idx_v.at[j]
            sl = pl.ds(j * _CHUNK, _CHUNK)
            copies.append(pltpu.async_copy(s_hbm.at[idx_j], s_v.at[sl], sem))
            copies.append(pltpu.async_copy(a_hbm.at[idx_j], a_v.at[sl], sem))
            copies.append(pltpu.async_copy(r_hbm.at[idx_j], r_v.at[sl], sem))
            copies.append(pltpu.async_copy(sn_hbm.at[idx_j], sn_v.at[sl], sem))
            copies.append(pltpu.async_copy(dw_hbm.at[idx_j], dw_v.at[sl], sem))
        for c in copies:
            c.wait()
        osl = pl.ds(base, _B_PER_W)
        pltpu.sync_copy(s_v, s_out.at[osl])
        pltpu.sync_copy(a_v, a_out.at[osl])
        pltpu.sync_copy(r_v, r_out.at[osl])
        pltpu.sync_copy(sn_v, sn_out.at[osl])
        pltpu.sync_copy(dw_v, dw_out.at[osl])

    return sample


_sample = _make_sample_kernel()


def kernel(s, a, r, s_next, dw, ind):
    ind3 = ind.reshape(_NW, _NCHUNK, _CHUNK)
    s_b, a_b, r_b, sn_b, dw_b = _sample(
        s, a.reshape(_MAX_SIZE), r.reshape(_MAX_SIZE), s_next,
        dw.reshape(_MAX_SIZE), ind3)
    return (s_b, a_b.reshape(_BATCH, 1), r_b.reshape(_BATCH, 1),
            sn_b, dw_b.reshape(_BATCH, 1))
